# baseline (speedup 1.0000x reference)
import jax
import jax.numpy as jnp
from jax.experimental import pallas as pl
from jax.experimental.pallas import tpu as pltpu

_STRIDES = (8, 16, 32, 64, 128)
_RANGES = ((0.0, 64.0), (64.0, 128.0), (128.0, 256.0), (256.0, 512.0), (512.0, 1e8))
_SIZES = ((100, 128), (50, 64), (25, 32), (13, 16), (7, 8))
_ALPHA = 0.25
_B, _C, _G = 8, 80, 32


def _fused_body(labels_ref, *refs):
    conf_refs = refs[0:5]
    loc_refs = refs[5:10]
    cen_refs = refs[10:15]
    out_ref = refs[15]
    b = pl.program_id(0)

    lc = 0.0
    ll = 0.0
    lctr = 0.0
    poses = 0.0
    for lvl in range(5):
        H, W = _SIZES[lvl]
        stride = float(_STRIDES[lvl])
        lo, hi = _RANGES[lvl]
        Y = (jax.lax.broadcasted_iota(jnp.int32, (H, W), 0)
             .astype(jnp.float32) + 0.5) * stride
        X = (jax.lax.broadcasted_iota(jnp.int32, (H, W), 1)
             .astype(jnp.float32) + 0.5) * stride

        inf = jnp.float32(jnp.inf)
        best_area = jnp.full((H, W), inf, jnp.float32)
        best_l = jnp.ones((H, W), jnp.float32)
        best_t = jnp.ones((H, W), jnp.float32)
        best_r = jnp.ones((H, W), jnp.float32)
        best_b = jnp.ones((H, W), jnp.float32)
        best_cls = jnp.full((H, W), -1.0, jnp.float32)
        for g in range(_G):
            cls_g = labels_ref[b, g, 0]
            x1 = labels_ref[b, g, 1]
            y1 = labels_ref[b, g, 2]
            x2 = labels_ref[b, g, 3]
            y2 = labels_ref[b, g, 4]
            area = (x2 - x1) * (y2 - y1)
            l = X - x1
            t = Y - y1
            r = x2 - X
            bb = y2 - Y
            mn = jnp.minimum(jnp.minimum(l, t), jnp.minimum(r, bb))
            m = mn > 0.0
            if lo > 0.0 or hi < 2048.0:
                mx = jnp.maximum(jnp.maximum(l, t), jnp.maximum(r, bb))
                if lo > 0.0:
                    m = m & (mx >= lo)
                if hi < 2048.0:
                    m = m & (mx <= hi)
            upd = m & (area < best_area)
            best_area = jnp.where(upd, area, best_area)
            best_l = jnp.where(upd, l, best_l)
            best_t = jnp.where(upd, t, best_t)
            best_r = jnp.where(upd, r, best_r)
            best_b = jnp.where(upd, bb, best_b)
            best_cls = jnp.where(upd, cls_g, best_cls)
        pos = best_cls >= 0.0
        tagf = best_cls

        loc = loc_refs[lvl][0]
        px1 = X - loc[0]
        py1 = Y - loc[1]
        px2 = X + loc[2]
        py2 = Y + loc[3]
        gx1 = X - best_l
        gy1 = Y - best_t
        gx2 = X + best_r
        gy2 = Y + best_b
        iw = jnp.maximum(jnp.minimum(px2, gx2) - jnp.maximum(px1, gx1), 0.0)
        ih = jnp.maximum(jnp.minimum(py2, gy2) - jnp.maximum(py1, gy1), 0.0)
        inter = iw * ih
        union = (px2 - px1) * (py2 - py1) + (gx2 - gx1) * (gy2 - gy1) - inter
        iou = inter / jnp.maximum(union, 1e-8)
        liou = -jnp.log(jnp.clip(iou, 1e-8, 1.0))
        ll = ll + jnp.sum(jnp.where(pos, liou, 0.0))

        lr = jnp.clip(jnp.minimum(best_l, best_r), 1e-6, None) / jnp.clip(
            jnp.maximum(best_l, best_r), 1e-6, None)
        tb = jnp.clip(jnp.minimum(best_t, best_b), 1e-6, None) / jnp.clip(
            jnp.maximum(best_t, best_b), 1e-6, None)
        ctr = jnp.sqrt(jnp.clip(lr * tb, 1e-6, 1.0))
        cenc = cen_refs[lvl][0, 0]
        bce = -(ctr * jnp.log(cenc) + (1.0 - ctr) * jnp.log(1.0 - cenc))
        lctr = lctr + jnp.sum(jnp.where(pos, bce, 0.0))
        poses = poses + jnp.sum(jnp.where(pos, 1.0, 0.0))

        c = conf_refs[lvl][0]
        cls_iota = jax.lax.broadcasted_iota(
            jnp.int32, (_C, H, W), 0).astype(jnp.float32)
        onehot = cls_iota == tagf[None]
        negsum = jnp.sum(c * c * jnp.log(1.0 - c))
        ctag = jnp.sum(jnp.where(onehot, c, 0.0), axis=0)
        ct = jnp.where(pos, ctag, 0.5)
        post_t = -_ALPHA * (1.0 - ct) * (1.0 - ct) * jnp.log(ct)
        neg_t = -(1.0 - _ALPHA) * ct * ct * jnp.log(1.0 - ct)
        corr = jnp.sum(jnp.where(pos, post_t - neg_t, 0.0))
        lc = lc + (-(1.0 - _ALPHA)) * negsum + corr

    lane = jax.lax.broadcasted_iota(jnp.int32, (1, 1, 128), 2)
    vec = (jnp.where(lane == 0, lc, 0.0)
           + jnp.where(lane == 1, ll, 0.0)
           + jnp.where(lane == 2, lctr, 0.0)
           + jnp.where(lane == 3, poses, 0.0))
    out_ref[...] = vec.astype(jnp.float32)


def kernel(conf0, conf1, conf2, conf3, conf4, loc0, loc1, loc2, loc3, loc4,
           cen0, cen1, cen2, cen3, cen4, labels):
    confs = (conf0, conf1, conf2, conf3, conf4)
    locs = (loc0, loc1, loc2, loc3, loc4)
    cens = (cen0, cen1, cen2, cen3, cen4)

    in_specs = [pl.BlockSpec(memory_space=pltpu.SMEM)]
    for i in range(5):
        H, W = _SIZES[i]
        in_specs.append(pl.BlockSpec((1, _C, H, W), lambda b: (b, 0, 0, 0)))
    for i in range(5):
        H, W = _SIZES[i]
        in_specs.append(pl.BlockSpec((1, 4, H, W), lambda b: (b, 0, 0, 0)))
    for i in range(5):
        H, W = _SIZES[i]
        in_specs.append(pl.BlockSpec((1, 1, H, W), lambda b: (b, 0, 0, 0)))

    out = pl.pallas_call(
        _fused_body,
        grid=(_B,),
        in_specs=in_specs,
        out_specs=pl.BlockSpec((1, 1, 128), lambda b: (b, 0, 0)),
        out_shape=jax.ShapeDtypeStruct((_B, 1, 128), jnp.float32),
        compiler_params=pltpu.CompilerParams(
            dimension_semantics=("parallel",)),
    )(labels, *confs, *locs, *cens)

    lc = out[:, 0, 0]
    ll = out[:, 0, 1]
    lctr = out[:, 0, 2]
    poses = out[:, 0, 3]
    per = jnp.where(poses > 0, lctr + (lc + ll) / jnp.maximum(poses, 1.0),
                    lctr + lc + ll)
    return jnp.mean(per)
